# baseline (speedup 1.0000x reference)
import functools

import jax
import jax.numpy as jnp
from jax import lax
from jax.experimental import pallas as pl
from jax.experimental.pallas import tpu as pltpu
from jax.experimental.pallas import tpu_sc as plsc

_NC = 2
_NS = 16
_NW = _NC * _NS
_CHUNK = 104
_GRP = 4
_NSET = 2


def _gather_kernel(B, D, n_chunks):
    mesh = plsc.VectorSubcoreMesh(core_axis_name="c", subcore_axis_name="s")
    b_per_w = B // _NW
    n_groups = n_chunks // _GRP

    @functools.partial(
        pl.kernel,
        out_type=jax.ShapeDtypeStruct((B, D), jnp.float32),
        mesh=mesh,
        scratch_types=[
            pltpu.VMEM((n_chunks, _CHUNK), jnp.int32),
            pltpu.VMEM((_NSET, _GRP, _CHUNK, D), jnp.float32),
            pltpu.SemaphoreType.DMA,
            pltpu.SemaphoreType.DMA,
        ],
        compiler_params=pltpu.CompilerParams(
            use_tc_tiling_on_sc=False, skip_device_barrier=True),
    )
    def k(idx_hbm, table_hbm, out_hbm, idx_v, rows_v, gsem, osem):
        wid = lax.axis_index("s") * _NC + lax.axis_index("c")
        base = wid * b_per_w
        pltpu.sync_copy(idx_hbm.at[wid], idx_v)

        def fire_gathers(g, s):
            for c in range(_GRP):
                pltpu.async_copy(
                    table_hbm.at[idx_v.at[g * _GRP + c]], rows_v.at[s, c], gsem)

        def drain(sem, s):
            for c in range(_GRP):
                pltpu.make_async_copy(
                    out_hbm.at[pl.ds(base, _CHUNK)], rows_v.at[s, c], sem).wait()

        def fire_stores(g, s):
            for c in range(_GRP):
                pltpu.async_copy(
                    rows_v.at[s, c],
                    out_hbm.at[pl.ds(base + (g * _GRP + c) * _CHUNK, _CHUNK)],
                    osem)

        fire_gathers(0, 0)

        @pl.loop(0, n_groups, step=_NSET)
        def _(g0):
            for sset in range(_NSET):
                g = g0 + sset
                s = sset
                drain(gsem, s)

                @pl.when(g >= 1)
                def _():
                    drain(osem, 1 - s)

                @pl.when(g + 1 < n_groups)
                def _():
                    fire_gathers(g + 1, 1 - s)

                fire_stores(g, s)

        drain(osem, (n_groups - 1) % _NSET)

    return k


def kernel(weights, indices):
    V, D = weights.shape
    R, K = indices.shape
    B = R * K
    n_chunks = B // (_NW * _CHUNK)
    idx = indices.reshape(_NW, n_chunks, _CHUNK).astype(jnp.int32)
    out = _gather_kernel(B, D, n_chunks)(idx, weights)
    return out.reshape(R, K, D)
